# Initial kernel scaffold; baseline (speedup 1.0000x reference)
import jax
import jax.numpy as jnp
from jax.experimental import pallas as pl


def kernel(x, emb_tables, lin_tables, bias):
    raise NotImplementedError("write your pallas kernel here")


# speedup vs baseline: 1.5562x; 1.5562x over previous
import functools

import jax
import jax.numpy as jnp
from jax import lax
from jax.experimental import pallas as pl
from jax.experimental.pallas import tpu as pltpu
from jax.experimental.pallas import tpu_sc as plsc

_NC = 2
_NS = 16
_NW = _NC * _NS

_IDX_PER_ROW = 104


def _fm_sc(xf, xfl, emb_flat, lin_flat, bias_vec, *, B, F, D):
    spw = B // _NW
    rows_pw = spw * F
    idx_rows_pw = rows_pw // _IDX_PER_ROW
    groups = spw // 16
    idx_rows_pg = (16 * F) // _IDX_PER_ROW
    gsz = 16 * F
    h = D // 2

    mesh = plsc.VectorSubcoreMesh(core_axis_name="c", subcore_axis_name="s")

    @functools.partial(
        pl.kernel,
        out_type=jax.ShapeDtypeStruct((B,), jnp.float32),
        mesh=mesh,
        compiler_params=pltpu.CompilerParams(
            needs_layout_passes=False, use_tc_tiling_on_sc=False),
        scratch_types=[
            pltpu.VMEM((idx_rows_pw, _IDX_PER_ROW), jnp.int32),
            pltpu.VMEM((idx_rows_pw, _IDX_PER_ROW), jnp.int32),
            pltpu.VMEM((gsz, D), jnp.float32),
            pltpu.VMEM((gsz,), jnp.float32),
            pltpu.VMEM((spw,), jnp.float32),
            pltpu.VMEM((16,), jnp.float32),
            pltpu.SemaphoreType.DMA,
        ],
    )
    def fm(xf_hbm, xfl_hbm, emb_hbm, lin_hbm, bias_hbm, out_hbm,
           idx_v, idxl_v, rows_v, lin_v, out_v, bias_v, sem):
        wid = lax.axis_index("s") * _NC + lax.axis_index("c")
        pltpu.sync_copy(xf_hbm.at[pl.ds(wid * idx_rows_pw, idx_rows_pw)], idx_v)
        pltpu.sync_copy(xfl_hbm.at[pl.ds(wid * idx_rows_pw, idx_rows_pw)], idxl_v)
        pltpu.sync_copy(bias_hbm, bias_v)

        iota16 = lax.iota(jnp.int32, 16)

        def group_body(g, carry):
            cps = []
            for q in range(idx_rows_pg):
                row = g * idx_rows_pg + q
                cps.append(pltpu.async_copy(
                    emb_hbm.at[idx_v.at[row]],
                    rows_v.at[pl.ds(q * _IDX_PER_ROW, _IDX_PER_ROW)], sem))
                cps.append(pltpu.async_copy(
                    lin_hbm.at[idxl_v.at[row]],
                    lin_v.at[pl.ds(q * _IDX_PER_ROW, _IDX_PER_ROW)], sem))
            for cp in cps:
                cp.wait()

            fo = lin_v[pl.ds(0, 16)]
            for f in range(1, F):
                fo = fo + lin_v[pl.ds(f * 16, 16)]

            def sample_body(l, sec_vec):
                rb = l * F
                s0 = rows_v[rb, pl.ds(0, 16)]
                s1 = rows_v[rb, pl.ds(h, 16)]
                q0 = s0 * s0
                q1 = s1 * s1
                for f in range(1, F):
                    e0 = rows_v[rb + f, pl.ds(0, 16)]
                    e1 = rows_v[rb + f, pl.ds(h, 16)]
                    s0 = s0 + e0
                    s1 = s1 + e1
                    q0 = q0 + e0 * e0
                    q1 = q1 + e1 * e1
                u = s0 * s0 + s1 * s1 - q0 - q1
                sec = 0.5 * jnp.sum(u)
                return jnp.where(iota16 == l, sec, sec_vec)

            sec_vec = lax.fori_loop(
                0, 16, sample_body, jnp.zeros((16,), jnp.float32))

            z = bias_v[...] + fo + sec_vec
            y = 1.0 / (1.0 + jnp.exp(-z))
            out_v[pl.ds(g * 16, 16)] = y
            return carry

        lax.fori_loop(0, groups, group_body, 0)
        pltpu.sync_copy(out_v, out_hbm.at[pl.ds(wid * spw, spw)])

    return fm(xf, xfl, emb_flat, lin_flat, bias_vec)


def kernel(x, emb_tables, lin_tables, bias):
    B, F = x.shape
    _, V, D = emb_tables.shape
    assert B % (8 * _NW) == 0 and (B // _NW) % 16 == 0
    assert (16 * F) % _IDX_PER_ROW == 0

    emb_flat = emb_tables.reshape(F * V, D)
    lin_flat = lin_tables.reshape(F * V)
    offs = (jnp.arange(F, dtype=jnp.int32) * V)[None, :]
    x_off = x + offs
    nrows = (B * F) // _IDX_PER_ROW
    xf = x_off.reshape(nrows, _IDX_PER_ROW)
    xfl = x_off.reshape(B // 16, 16, F).transpose(0, 2, 1).reshape(
        nrows, _IDX_PER_ROW)
    bias_vec = jnp.broadcast_to(bias.astype(jnp.float32), (16,))

    out = _fm_sc(xf, xfl, emb_flat, lin_flat, bias_vec, B=B, F=F, D=D)
    return out.reshape(B, 1)


# speedup vs baseline: 1.5718x; 1.0100x over previous
import functools

import jax
import jax.numpy as jnp
from jax import lax
from jax.experimental import pallas as pl
from jax.experimental.pallas import tpu as pltpu
from jax.experimental.pallas import tpu_sc as plsc

_NC = 2
_NS = 16
_NW = _NC * _NS

_IDX_PER_ROW = 104


def _fm_sc(xf, emb_flat, lin_flat, bias_vec, *, B, F, D):
    spw = B // _NW
    rows_pw = spw * F
    idx_rows_pw = rows_pw // _IDX_PER_ROW
    groups = spw // 16
    idx_rows_pg = (16 * F) // _IDX_PER_ROW
    gsz = 16 * F
    h = D // 2

    mesh = plsc.VectorSubcoreMesh(core_axis_name="c", subcore_axis_name="s")

    @functools.partial(
        pl.kernel,
        out_type=jax.ShapeDtypeStruct((B,), jnp.float32),
        mesh=mesh,
        compiler_params=pltpu.CompilerParams(
            needs_layout_passes=False, use_tc_tiling_on_sc=False),
        scratch_types=[
            pltpu.VMEM((idx_rows_pw, _IDX_PER_ROW), jnp.int32),
            pltpu.VMEM((gsz, D), jnp.float32),
            pltpu.VMEM((gsz + 32, ), jnp.float32),
            pltpu.VMEM((spw,), jnp.float32),
            pltpu.VMEM((16,), jnp.float32),
            pltpu.SemaphoreType.DMA,
        ],
    )
    def fm(xf_hbm, emb_hbm, lin_hbm, bias_hbm, out_hbm,
           idx_v, rows_v, lin_v, out_v, bias_v, sem):
        wid = lax.axis_index("s") * _NC + lax.axis_index("c")
        pltpu.sync_copy(xf_hbm.at[pl.ds(wid * idx_rows_pw, idx_rows_pw)], idx_v)
        pltpu.sync_copy(bias_hbm, bias_v)

        iota16 = lax.iota(jnp.int32, 16)

        def group_body(g, carry):
            cps = []
            for q in range(idx_rows_pg):
                row = g * idx_rows_pg + q
                cps.append(pltpu.async_copy(
                    emb_hbm.at[idx_v.at[row]],
                    rows_v.at[pl.ds(q * _IDX_PER_ROW, _IDX_PER_ROW)], sem))
                cps.append(pltpu.async_copy(
                    lin_hbm.at[idx_v.at[row]],
                    lin_v.at[pl.ds(q * _IDX_PER_ROW, _IDX_PER_ROW)], sem))
            for cp in cps:
                cp.wait()

            def sample_body(l, sec_vec):
                rb = l * F
                s0 = rows_v[rb, pl.ds(0, 16)]
                s1 = rows_v[rb, pl.ds(h, 16)]
                q0 = s0 * s0
                q1 = s1 * s1
                for f in range(1, F):
                    e0 = rows_v[rb + f, pl.ds(0, 16)]
                    e1 = rows_v[rb + f, pl.ds(h, 16)]
                    s0 = s0 + e0
                    s1 = s1 + e1
                    q0 = q0 + e0 * e0
                    q1 = q1 + e1 * e1
                u = s0 * s0 + s1 * s1 - q0 - q1
                l1 = lin_v[pl.ds(rb, 16)]
                l2 = jnp.where(iota16 < F - 16,
                               lin_v[pl.ds(rb + 16, 16)], 0.0)
                sec = jnp.sum(0.5 * u + l1 + l2)
                return jnp.where(iota16 == l, sec, sec_vec)

            sec_vec = lax.fori_loop(
                0, 16, sample_body, jnp.zeros((16,), jnp.float32))

            z = bias_v[...] + sec_vec
            y = 1.0 / (1.0 + jnp.exp(-z))
            out_v[pl.ds(g * 16, 16)] = y
            return carry

        lax.fori_loop(0, groups, group_body, 0)
        pltpu.sync_copy(out_v, out_hbm.at[pl.ds(wid * spw, spw)])

    return fm(xf, emb_flat, lin_flat, bias_vec)


def kernel(x, emb_tables, lin_tables, bias):
    B, F = x.shape
    _, V, D = emb_tables.shape
    assert B % (8 * _NW) == 0 and (B // _NW) % 16 == 0
    assert (16 * F) % _IDX_PER_ROW == 0

    emb_flat = emb_tables.reshape(F * V, D)
    lin_flat = lin_tables.reshape(F * V)
    offs = (jnp.arange(F, dtype=jnp.int32) * V)[None, :]
    x_off = x + offs
    nrows = (B * F) // _IDX_PER_ROW
    xf = x_off.reshape(nrows, _IDX_PER_ROW)
    bias_vec = jnp.broadcast_to(bias.astype(jnp.float32), (16,))

    out = _fm_sc(xf, emb_flat, lin_flat, bias_vec, B=B, F=F, D=D)
    return out.reshape(B, 1)
